# baseline (speedup 1.0000x reference)
import jax
import jax.numpy as jnp
from jax import lax
from jax.experimental import pallas as pl
from jax.experimental.pallas import tpu as pltpu
from jax.experimental.pallas import tpu_sc as plsc

VOCAB = 100000
DIM = 128
N_CACHE = 16384
SMOOTH = 0.2

NC = 2
NS = 16
NW = NC * NS
B_PER_W = N_CACHE // NW
CHUNK = 128
N_CHUNK = B_PER_W // CHUNK

VOCAB_PAD = 100352
BINS_PER_SUB = VOCAB_PAD // NS


def _lanesum(x, lanes):
    for k in (1, 2, 4, 8):
        x = x + jnp.take(x, lanes ^ k, mode="promise_in_bounds")
    return x


def _rdist(s):
    xi = plsc.bitcast(s, jnp.int32)
    yi = jnp.int32(0x5F3759DF) - (xi >> 1)
    r = plsc.bitcast(yi, jnp.float32)
    for _ in range(3):
        r = r * (1.5 - 0.5 * s * r * r)
    return s * r


def _fused_body(h_hbm, ch_hbm, idx_hbm, out_hbm,
                idx_v, g0, g1, c0, c1, kv_v, zbuf, bins,
                sem0, sem1, csem0, csem1):
    c = lax.axis_index("c")
    s = lax.axis_index("s")
    wid = s * NC + c
    base_row = wid * N_CHUNK

    pltpu.sync_copy(idx_hbm.at[pl.ds(base_row, N_CHUNK)], idx_v)

    gbufs = [g0, g1]
    cbufs = [c0, c1]
    gsems = [sem0, sem1]
    csems = [csem0, csem1]

    d_g = [None, None]
    d_c = [None, None]
    d_g[0] = pltpu.async_copy(h_hbm.at[idx_v.at[0]], g0, sem0)
    d_c[0] = pltpu.async_copy(
        ch_hbm.at[pl.ds(pl.multiple_of(wid * B_PER_W, CHUNK), CHUNK)], c0, csem0)

    def zb(i, carry):
        zbuf[pl.ds(pl.multiple_of(i * 16, 16), 16)] = jnp.zeros((16,), jnp.float32)
        return carry
    lax.fori_loop(0, BINS_PER_SUB // 16, zb, 0)
    pltpu.sync_copy(zbuf, bins.at[pl.ds(s * BINS_PER_SUB, BINS_PER_SUB)])
    plsc.subcore_barrier()

    lanes = lax.iota(jnp.int32, 16)

    for j in range(N_CHUNK):
        cur = j % 2
        nxt = (j + 1) % 2
        if j + 1 < N_CHUNK:
            d_g[nxt] = pltpu.async_copy(h_hbm.at[idx_v.at[j + 1]], gbufs[nxt],
                                        gsems[nxt])
            d_c[nxt] = pltpu.async_copy(
                ch_hbm.at[pl.ds(pl.multiple_of(wid * B_PER_W + (j + 1) * CHUNK,
                                               CHUNK), CHUNK)],
                cbufs[nxt], csems[nxt])
        d_g[cur].wait()
        d_c[cur].wait()
        gb = gbufs[cur]
        cb = cbufs[cur]

        @plsc.parallel_loop(0, CHUNK // 16, 1, unroll=1)
        def group(g):
            rbase = pl.multiple_of(g * 16, 16)
            vals = []
            for r in range(16):
                row = rbase + r
                acc0 = jnp.zeros((16,), jnp.float32)
                acc1 = jnp.zeros((16,), jnp.float32)
                for v in range(4):
                    dv = cb[row, pl.ds(v * 16, 16)] - gb[row, pl.ds(v * 16, 16)]
                    acc0 = acc0 + dv * dv
                for v in range(4, 8):
                    dv = cb[row, pl.ds(v * 16, 16)] - gb[row, pl.ds(v * 16, 16)]
                    acc1 = acc1 + dv * dv
                vals.append(jnp.where(lanes == r, jnp.sum(acc0 + acc1), 0.0))
            while len(vals) > 1:
                vals = [a + b for a, b in zip(vals[::2], vals[1::2])]
            kvvec = jnp.exp(_rdist(vals[0]) * jnp.float32(1.0 / SMOOTH))
            kv_v[pl.ds(pl.multiple_of(j * CHUNK + g * 16, 16), 16)] = kvvec

    for j in range(N_CHUNK):
        pltpu.sync_copy(kv_v.at[pl.ds(j * CHUNK, CHUNK)],
                        bins.at[idx_v.at[j]], add=True)
    plsc.subcore_barrier()

    pltpu.sync_copy(bins.at[pl.ds(s * BINS_PER_SUB, BINS_PER_SUB)], zbuf)
    pltpu.sync_copy(zbuf, out_hbm.at[c, pl.ds(s * BINS_PER_SUB, BINS_PER_SUB)])


_fused_call = pl.kernel(
    _fused_body,
    out_type=jax.ShapeDtypeStruct((NC, VOCAB_PAD), jnp.float32),
    mesh=plsc.VectorSubcoreMesh(core_axis_name="c", subcore_axis_name="s"),
    compiler_params=pltpu.CompilerParams(needs_layout_passes=False),
    scratch_types=[
        pltpu.VMEM((N_CHUNK, 128), jnp.int32),
        pltpu.VMEM((CHUNK, DIM), jnp.float32),
        pltpu.VMEM((CHUNK, DIM), jnp.float32),
        pltpu.VMEM((CHUNK, DIM), jnp.float32),
        pltpu.VMEM((CHUNK, DIM), jnp.float32),
        pltpu.VMEM((B_PER_W,), jnp.float32),
        pltpu.VMEM((BINS_PER_SUB,), jnp.float32),
        pltpu.VMEM_SHARED((VOCAB_PAD,), jnp.float32),
        pltpu.SemaphoreType.DMA,
        pltpu.SemaphoreType.DMA,
        pltpu.SemaphoreType.DMA,
        pltpu.SemaphoreType.DMA,
    ],
)


def _softmax_body(b_ref, o_ref):
    xb = b_ref[...]
    x = xb[0:1, :] + xb[1:2, :]
    col = lax.broadcasted_iota(jnp.int32, (1, VOCAB_PAD), 1)
    valid = col < VOCAB
    neg = jnp.float32(-jnp.inf)
    m = jnp.max(jnp.where(valid, x, neg), axis=1, keepdims=True)
    sub = x - m
    e = jnp.where(valid, jnp.exp(sub), 0.0)
    lse = jnp.log(jnp.sum(e, axis=1, keepdims=True))
    o_ref[...] = (sub - lse)[:, :VOCAB]


def _softmax_call(partial):
    return pl.pallas_call(
        _softmax_body,
        out_shape=jax.ShapeDtypeStruct((1, VOCAB), jnp.float32),
    )(partial)


def kernel(h_t, cache_hiddens, cache_items):
    idx = cache_items.astype(jnp.int32).reshape(N_CACHE // 128, 128)
    partial = _fused_call(h_t, cache_hiddens, idx)
    return _softmax_call(partial)


# speedup vs baseline: 1.3296x; 1.3296x over previous
import jax
import jax.numpy as jnp
from jax import lax
from jax.experimental import pallas as pl
from jax.experimental.pallas import tpu as pltpu
from jax.experimental.pallas import tpu_sc as plsc

VOCAB = 100000
DIM = 128
N_CACHE = 16384
SMOOTH = 0.2

NC = 2
NS = 16
NW = NC * NS
B_PER_W = N_CACHE // NW
CHUNK = 128
N_CHUNK = B_PER_W // CHUNK

VOCAB_PAD = 100352
BINS_PER_SUB = VOCAB_PAD // NS


def _lanesum(x, lanes):
    for k in (1, 2, 4, 8):
        x = x + jnp.take(x, lanes ^ k, mode="promise_in_bounds")
    return x


def _rdist(s):
    xi = plsc.bitcast(s, jnp.int32)
    yi = jnp.int32(0x5F3759DF) - (xi >> 1)
    r = plsc.bitcast(yi, jnp.float32)
    for _ in range(3):
        r = r * (1.5 - 0.5 * s * r * r)
    return s * r


def _fused_body(h_hbm, ch_hbm, idx_hbm, z_hbm, out_hbm,
                idx_v, g0, g1, c0, c1, kv_v, bins,
                sem0, sem1, csem0, csem1):
    c = lax.axis_index("c")
    s = lax.axis_index("s")
    wid = s * NC + c
    base_row = wid * N_CHUNK

    pltpu.sync_copy(idx_hbm.at[pl.ds(base_row, N_CHUNK)], idx_v)

    gbufs = [g0, g1]
    cbufs = [c0, c1]
    gsems = [sem0, sem1]
    csems = [csem0, csem1]

    d_g = [None, None]
    d_c = [None, None]
    d_g[0] = pltpu.async_copy(h_hbm.at[idx_v.at[0]], g0, sem0)
    d_c[0] = pltpu.async_copy(
        ch_hbm.at[pl.ds(pl.multiple_of(wid * B_PER_W, CHUNK), CHUNK)], c0, csem0)

    pltpu.sync_copy(z_hbm.at[pl.ds(s * BINS_PER_SUB, BINS_PER_SUB)],
                    bins.at[pl.ds(s * BINS_PER_SUB, BINS_PER_SUB)])
    plsc.subcore_barrier()

    lanes = lax.iota(jnp.int32, 16)

    for j in range(N_CHUNK):
        cur = j % 2
        nxt = (j + 1) % 2
        if j + 1 < N_CHUNK:
            d_g[nxt] = pltpu.async_copy(h_hbm.at[idx_v.at[j + 1]], gbufs[nxt],
                                        gsems[nxt])
            d_c[nxt] = pltpu.async_copy(
                ch_hbm.at[pl.ds(pl.multiple_of(wid * B_PER_W + (j + 1) * CHUNK,
                                               CHUNK), CHUNK)],
                cbufs[nxt], csems[nxt])
        d_g[cur].wait()
        d_c[cur].wait()
        gb = gbufs[cur]
        cb = cbufs[cur]

        @plsc.parallel_loop(0, CHUNK // 16, 1, unroll=2)
        def group(g):
            rbase = pl.multiple_of(g * 16, 16)
            vals = []
            for r in range(16):
                row = rbase + r
                acc0 = jnp.zeros((16,), jnp.float32)
                acc1 = jnp.zeros((16,), jnp.float32)
                for v in range(4):
                    dv = cb[row, pl.ds(v * 16, 16)] - gb[row, pl.ds(v * 16, 16)]
                    acc0 = acc0 + dv * dv
                for v in range(4, 8):
                    dv = cb[row, pl.ds(v * 16, 16)] - gb[row, pl.ds(v * 16, 16)]
                    acc1 = acc1 + dv * dv
                vals.append(jnp.where(lanes == r, jnp.sum(acc0 + acc1), 0.0))
            while len(vals) > 1:
                vals = [a + b for a, b in zip(vals[::2], vals[1::2])]
            kvvec = jnp.exp(_rdist(vals[0]) * jnp.float32(1.0 / SMOOTH))
            kv_v[pl.ds(pl.multiple_of(j * CHUNK + g * 16, 16), 16)] = kvvec

    for j in range(N_CHUNK):
        pltpu.sync_copy(kv_v.at[pl.ds(j * CHUNK, CHUNK)],
                        bins.at[idx_v.at[j]], add=True)
    plsc.subcore_barrier()

    pltpu.sync_copy(bins.at[pl.ds(s * BINS_PER_SUB, BINS_PER_SUB)],
                    out_hbm.at[c, pl.ds(s * BINS_PER_SUB, BINS_PER_SUB)])


_fused_call = pl.kernel(
    _fused_body,
    out_type=jax.ShapeDtypeStruct((NC, VOCAB_PAD), jnp.float32),
    mesh=plsc.VectorSubcoreMesh(core_axis_name="c", subcore_axis_name="s"),
    compiler_params=pltpu.CompilerParams(needs_layout_passes=False),
    scratch_types=[
        pltpu.VMEM((N_CHUNK, 128), jnp.int32),
        pltpu.VMEM((CHUNK, DIM), jnp.float32),
        pltpu.VMEM((CHUNK, DIM), jnp.float32),
        pltpu.VMEM((CHUNK, DIM), jnp.float32),
        pltpu.VMEM((CHUNK, DIM), jnp.float32),
        pltpu.VMEM((B_PER_W,), jnp.float32),
        pltpu.VMEM_SHARED((VOCAB_PAD,), jnp.float32),
        pltpu.SemaphoreType.DMA,
        pltpu.SemaphoreType.DMA,
        pltpu.SemaphoreType.DMA,
        pltpu.SemaphoreType.DMA,
    ],
)


def _softmax_body(b_ref, o_ref):
    xb = b_ref[...]
    x = xb[0:1, :] + xb[1:2, :]
    col = lax.broadcasted_iota(jnp.int32, (1, VOCAB_PAD), 1)
    valid = col < VOCAB
    neg = jnp.float32(-jnp.inf)
    m = jnp.max(jnp.where(valid, x, neg), axis=1, keepdims=True)
    sub = x - m
    e = jnp.where(valid, jnp.exp(sub), 0.0)
    lse = jnp.log(jnp.sum(e, axis=1, keepdims=True))
    o_ref[...] = (sub - lse)[:, :VOCAB]


def _softmax_call(partial):
    return pl.pallas_call(
        _softmax_body,
        out_shape=jax.ShapeDtypeStruct((1, VOCAB), jnp.float32),
    )(partial)


def kernel(h_t, cache_hiddens, cache_items):
    idx = cache_items.astype(jnp.int32).reshape(N_CACHE // 128, 128)
    zeros = jnp.zeros((VOCAB_PAD,), jnp.float32)
    partial = _fused_call(h_t, cache_hiddens, idx, zeros)
    return _softmax_call(partial)


# speedup vs baseline: 1.3601x; 1.0229x over previous
import jax
import jax.numpy as jnp
from jax import lax
from jax.experimental import pallas as pl
from jax.experimental.pallas import tpu as pltpu
from jax.experimental.pallas import tpu_sc as plsc

VOCAB = 100000
DIM = 128
N_CACHE = 16384
SMOOTH = 0.2

NC = 2
NS = 16
NW = NC * NS
B_PER_W = N_CACHE // NW
CHUNK = 128
N_CHUNK = B_PER_W // CHUNK

VOCAB_PAD = 100352
BINS_PER_SUB = VOCAB_PAD // NS


def _lanesum(x, lanes):
    for k in (1, 2, 4, 8):
        x = x + jnp.take(x, lanes ^ k, mode="promise_in_bounds")
    return x


def _rdist(s):
    xi = plsc.bitcast(s, jnp.int32)
    yi = jnp.int32(0x5F3759DF) - (xi >> 1)
    r = plsc.bitcast(yi, jnp.float32)
    for _ in range(3):
        r = r * (1.5 - 0.5 * s * r * r)
    return s * r


def _fused_body(h_hbm, ch_hbm, idx_hbm, out_hbm,
                idx_v, g0, g1, c0, c1, kv_v, zbuf, bins,
                sem0, sem1, csem0, csem1, ssem):
    c = lax.axis_index("c")
    s = lax.axis_index("s")
    wid = s * NC + c
    base_row = wid * N_CHUNK

    pltpu.sync_copy(idx_hbm.at[pl.ds(base_row, N_CHUNK)], idx_v)

    gbufs = [g0, g1]
    cbufs = [c0, c1]
    gsems = [sem0, sem1]
    csems = [csem0, csem1]

    d_g = [None, None]
    d_c = [None, None]
    d_g[0] = pltpu.async_copy(h_hbm.at[idx_v.at[0]], g0, sem0)
    d_c[0] = pltpu.async_copy(
        ch_hbm.at[pl.ds(pl.multiple_of(wid * B_PER_W, CHUNK), CHUNK)], c0, csem0)

    def zb(i, carry):
        zbuf[pl.ds(pl.multiple_of(i * 16, 16), 16)] = jnp.zeros((16,), jnp.float32)
        return carry
    lax.fori_loop(0, BINS_PER_SUB // 16, zb, 0)
    pltpu.sync_copy(zbuf, bins.at[pl.ds(s * BINS_PER_SUB, BINS_PER_SUB)])
    plsc.subcore_barrier()

    lanes = lax.iota(jnp.int32, 16)
    scat = []

    for j in range(N_CHUNK):
        cur = j % 2
        nxt = (j + 1) % 2
        if j + 1 < N_CHUNK:
            d_g[nxt] = pltpu.async_copy(h_hbm.at[idx_v.at[j + 1]], gbufs[nxt],
                                        gsems[nxt])
            d_c[nxt] = pltpu.async_copy(
                ch_hbm.at[pl.ds(pl.multiple_of(wid * B_PER_W + (j + 1) * CHUNK,
                                               CHUNK), CHUNK)],
                cbufs[nxt], csems[nxt])
        d_g[cur].wait()
        d_c[cur].wait()
        gb = gbufs[cur]
        cb = cbufs[cur]

        @plsc.parallel_loop(0, CHUNK // 16, 1, unroll=2)
        def group(g):
            rbase = pl.multiple_of(g * 16, 16)
            vals = []
            for r in range(16):
                row = rbase + r
                acc0 = jnp.zeros((16,), jnp.float32)
                acc1 = jnp.zeros((16,), jnp.float32)
                for v in range(4):
                    dv = cb[row, pl.ds(v * 16, 16)] - gb[row, pl.ds(v * 16, 16)]
                    acc0 = acc0 + dv * dv
                for v in range(4, 8):
                    dv = cb[row, pl.ds(v * 16, 16)] - gb[row, pl.ds(v * 16, 16)]
                    acc1 = acc1 + dv * dv
                vals.append(jnp.where(lanes == r, jnp.sum(acc0 + acc1), 0.0))
            while len(vals) > 1:
                vals = [a + b for a, b in zip(vals[::2], vals[1::2])]
            kvvec = jnp.exp(_rdist(vals[0]) * jnp.float32(1.0 / SMOOTH))
            kv_v[pl.ds(pl.multiple_of(j * CHUNK + g * 16, 16), 16)] = kvvec

        scat.append(pltpu.async_copy(kv_v.at[pl.ds(j * CHUNK, CHUNK)],
                                     bins.at[idx_v.at[j]], ssem, add=True))

    for d in scat:
        d.wait()
    plsc.subcore_barrier()

    pltpu.sync_copy(bins.at[pl.ds(s * BINS_PER_SUB, BINS_PER_SUB)],
                    out_hbm.at[c, pl.ds(s * BINS_PER_SUB, BINS_PER_SUB)])


_fused_call = pl.kernel(
    _fused_body,
    out_type=jax.ShapeDtypeStruct((NC, VOCAB_PAD), jnp.float32),
    mesh=plsc.VectorSubcoreMesh(core_axis_name="c", subcore_axis_name="s"),
    compiler_params=pltpu.CompilerParams(needs_layout_passes=False),
    scratch_types=[
        pltpu.VMEM((N_CHUNK, 128), jnp.int32),
        pltpu.VMEM((CHUNK, DIM), jnp.float32),
        pltpu.VMEM((CHUNK, DIM), jnp.float32),
        pltpu.VMEM((CHUNK, DIM), jnp.float32),
        pltpu.VMEM((CHUNK, DIM), jnp.float32),
        pltpu.VMEM((B_PER_W,), jnp.float32),
        pltpu.VMEM((BINS_PER_SUB,), jnp.float32),
        pltpu.VMEM_SHARED((VOCAB_PAD,), jnp.float32),
        pltpu.SemaphoreType.DMA,
        pltpu.SemaphoreType.DMA,
        pltpu.SemaphoreType.DMA,
        pltpu.SemaphoreType.DMA,
        pltpu.SemaphoreType.DMA,
    ],
)


def _softmax_body(b_ref, o_ref):
    xb = b_ref[...]
    x = xb[0:1, :] + xb[1:2, :]
    col = lax.broadcasted_iota(jnp.int32, (1, VOCAB_PAD), 1)
    valid = col < VOCAB
    neg = jnp.float32(-jnp.inf)
    m = jnp.max(jnp.where(valid, x, neg), axis=1, keepdims=True)
    sub = x - m
    e = jnp.where(valid, jnp.exp(sub), 0.0)
    lse = jnp.log(jnp.sum(e, axis=1, keepdims=True))
    o_ref[...] = (sub - lse)[:, :VOCAB]


def _softmax_call(partial):
    return pl.pallas_call(
        _softmax_body,
        out_shape=jax.ShapeDtypeStruct((1, VOCAB), jnp.float32),
    )(partial)


def kernel(h_t, cache_hiddens, cache_items):
    idx = cache_items.astype(jnp.int32).reshape(N_CACHE // 128, 128)
    partial = _fused_call(h_t, cache_hiddens, idx)
    return _softmax_call(partial)


# speedup vs baseline: 1.3772x; 1.0125x over previous
import jax
import jax.numpy as jnp
from jax import lax
from jax.experimental import pallas as pl
from jax.experimental.pallas import tpu as pltpu
from jax.experimental.pallas import tpu_sc as plsc

VOCAB = 100000
DIM = 128
N_CACHE = 16384
SMOOTH = 0.2

NC = 2
NS = 16
NW = NC * NS
B_PER_W = N_CACHE // NW
CHUNK = 128
N_CHUNK = B_PER_W // CHUNK

VOCAB_PAD = 100352
BINS_PER_SUB = VOCAB_PAD // NS


def _lanesum(x, lanes):
    for k in (1, 2, 4, 8):
        x = x + jnp.take(x, lanes ^ k, mode="promise_in_bounds")
    return x


def _rdist(s):
    xi = plsc.bitcast(s, jnp.int32)
    yi = jnp.int32(0x5F3759DF) - (xi >> 1)
    r = plsc.bitcast(yi, jnp.float32)
    for _ in range(3):
        r = r * (1.5 - 0.5 * s * r * r)
    return s * r


def _fused_body(h_hbm, ch_hbm, idx_hbm, out_hbm,
                idx_v, g0, g1, c0, c1, kv_v, zbuf, bins,
                sem0, sem1, csem0, csem1, ssem):
    c = lax.axis_index("c")
    s = lax.axis_index("s")
    wid = s * NC + c
    base_row = wid * N_CHUNK

    pltpu.sync_copy(idx_hbm.at[pl.ds(base_row, N_CHUNK)], idx_v)

    gbufs = [g0, g1]
    cbufs = [c0, c1]
    gsems = [sem0, sem1]
    csems = [csem0, csem1]

    d_g = [None, None]
    d_c = [None, None]
    d_g[0] = pltpu.async_copy(h_hbm.at[idx_v.at[0]], g0, sem0)
    d_c[0] = pltpu.async_copy(
        ch_hbm.at[pl.ds(pl.multiple_of(wid * B_PER_W, CHUNK), CHUNK)], c0, csem0)

    def zb(i, carry):
        zbuf[pl.ds(pl.multiple_of(i * 16, 16), 16)] = jnp.zeros((16,), jnp.float32)
        return carry
    lax.fori_loop(0, BINS_PER_SUB // 16, zb, 0)
    pltpu.sync_copy(zbuf, bins.at[pl.ds(s * BINS_PER_SUB, BINS_PER_SUB)])
    plsc.subcore_barrier()

    lanes = lax.iota(jnp.int32, 16)
    scat = []

    for j in range(N_CHUNK):
        cur = j % 2
        nxt = (j + 1) % 2
        if j + 1 < N_CHUNK:
            d_g[nxt] = pltpu.async_copy(h_hbm.at[idx_v.at[j + 1]], gbufs[nxt],
                                        gsems[nxt])
            d_c[nxt] = pltpu.async_copy(
                ch_hbm.at[pl.ds(pl.multiple_of(wid * B_PER_W + (j + 1) * CHUNK,
                                               CHUNK), CHUNK)],
                cbufs[nxt], csems[nxt])
        d_g[cur].wait()
        d_c[cur].wait()
        gb = gbufs[cur]
        cb = cbufs[cur]

        @plsc.parallel_loop(0, CHUNK // 16, 1, unroll=2)
        def group(g):
            rbase = pl.multiple_of(g * 16, 16)
            vals = []
            for r in range(16):
                row = rbase + r
                sq = []
                for v in range(8):
                    dv = cb[row, pl.ds(v * 16, 16)] - gb[row, pl.ds(v * 16, 16)]
                    sq.append(dv * dv)
                while len(sq) > 1:
                    sq = [a + b for a, b in zip(sq[::2], sq[1::2])]
                vals.append(jnp.where(lanes == r, jnp.sum(sq[0]), 0.0))
            while len(vals) > 1:
                vals = [a + b for a, b in zip(vals[::2], vals[1::2])]
            kvvec = jnp.exp(_rdist(vals[0]) * jnp.float32(1.0 / SMOOTH))
            kv_v[pl.ds(pl.multiple_of(j * CHUNK + g * 16, 16), 16)] = kvvec

        scat.append(pltpu.async_copy(kv_v.at[pl.ds(j * CHUNK, CHUNK)],
                                     bins.at[idx_v.at[j]], ssem, add=True))

    for d in scat:
        d.wait()
    plsc.subcore_barrier()

    pltpu.sync_copy(bins.at[pl.ds(s * BINS_PER_SUB, BINS_PER_SUB)],
                    out_hbm.at[c, pl.ds(s * BINS_PER_SUB, BINS_PER_SUB)])


_fused_call = pl.kernel(
    _fused_body,
    out_type=jax.ShapeDtypeStruct((NC, VOCAB_PAD), jnp.float32),
    mesh=plsc.VectorSubcoreMesh(core_axis_name="c", subcore_axis_name="s"),
    compiler_params=pltpu.CompilerParams(needs_layout_passes=False),
    scratch_types=[
        pltpu.VMEM((N_CHUNK, 128), jnp.int32),
        pltpu.VMEM((CHUNK, DIM), jnp.float32),
        pltpu.VMEM((CHUNK, DIM), jnp.float32),
        pltpu.VMEM((CHUNK, DIM), jnp.float32),
        pltpu.VMEM((CHUNK, DIM), jnp.float32),
        pltpu.VMEM((B_PER_W,), jnp.float32),
        pltpu.VMEM((BINS_PER_SUB,), jnp.float32),
        pltpu.VMEM_SHARED((VOCAB_PAD,), jnp.float32),
        pltpu.SemaphoreType.DMA,
        pltpu.SemaphoreType.DMA,
        pltpu.SemaphoreType.DMA,
        pltpu.SemaphoreType.DMA,
        pltpu.SemaphoreType.DMA,
    ],
)


def _softmax_body(b_ref, o_ref):
    xb = b_ref[...]
    x = xb[0:1, :] + xb[1:2, :]
    col = lax.broadcasted_iota(jnp.int32, (1, VOCAB_PAD), 1)
    valid = col < VOCAB
    neg = jnp.float32(-jnp.inf)
    m = jnp.max(jnp.where(valid, x, neg), axis=1, keepdims=True)
    sub = x - m
    e = jnp.where(valid, jnp.exp(sub), 0.0)
    lse = jnp.log(jnp.sum(e, axis=1, keepdims=True))
    o_ref[...] = (sub - lse)[:, :VOCAB]


def _softmax_call(partial):
    return pl.pallas_call(
        _softmax_body,
        out_shape=jax.ShapeDtypeStruct((1, VOCAB), jnp.float32),
    )(partial)


def kernel(h_t, cache_hiddens, cache_items):
    idx = cache_items.astype(jnp.int32).reshape(N_CACHE // 128, 128)
    partial = _fused_call(h_t, cache_hiddens, idx)
    return _softmax_call(partial)
